# baseline (device time: 222665 ns/iter reference)
import jax
import jax.numpy as jnp
from jax import lax
from jax.experimental import pallas as pl
from jax.experimental.pallas import tpu as pltpu

_N_CHUNKS = 16


def kernel(x):
    m, n = x.shape
    half = m // 2
    ch = half // _N_CHUNKS

    def body(
        x_ref,
        out_ref,
        recv_ref,
        stage_ref,
        stage_sems,
        cp_sems_o,
        send_sems_x,
        recv_sems_x,
        send_sems_y,
        recv_sems_y,
    ):
        my_x = lax.axis_index("x")
        my_y = lax.axis_index("y")
        x_nbr = (1 - my_x, my_y)
        y_nbr = (my_x, 1 - my_y)

        barrier = pltpu.get_barrier_semaphore()
        for nbr in (x_nbr, y_nbr):
            pl.semaphore_signal(
                barrier, inc=1, device_id=nbr,
                device_id_type=pl.DeviceIdType.MESH,
            )
        pl.semaphore_wait(barrier, 2)

        my_off = my_y * half

        stage_cps = []
        for k in range(_N_CHUNKS):
            cp = pltpu.make_async_copy(
                x_ref.at[pl.ds(my_off + k * ch, ch), :],
                stage_ref.at[pl.ds(k * ch, ch), :],
                stage_sems.at[k],
            )
            cp.start()
            stage_cps.append(cp)

        x_rdmas = []
        for k in range(_N_CHUNKS):
            stage_cps[k].wait()
            rdma = pltpu.make_async_remote_copy(
                src_ref=stage_ref.at[pl.ds(k * ch, ch), :],
                dst_ref=recv_ref.at[pl.ds(k * ch, ch), :],
                send_sem=send_sems_x.at[k],
                recv_sem=recv_sems_x.at[k],
                device_id=x_nbr,
                device_id_type=pl.DeviceIdType.MESH,
            )
            rdma.start()
            x_rdmas.append(rdma)

        y_rdmas = []
        out_cps = []
        for k in range(_N_CHUNKS):
            x_rdmas[k].wait_recv()
            rows = pl.ds(k * ch, ch)
            recv_ref[rows, :] = recv_ref[rows, :] + stage_ref[rows, :]
            rdma = pltpu.make_async_remote_copy(
                src_ref=recv_ref.at[rows, :],
                dst_ref=out_ref.at[pl.ds(my_off + k * ch, ch), :],
                send_sem=send_sems_y.at[k],
                recv_sem=recv_sems_y.at[k],
                device_id=y_nbr,
                device_id_type=pl.DeviceIdType.MESH,
            )
            rdma.start()
            y_rdmas.append(rdma)
            cp_o = pltpu.make_async_copy(
                recv_ref.at[rows, :],
                out_ref.at[pl.ds(my_off + k * ch, ch), :],
                cp_sems_o.at[k],
            )
            cp_o.start()
            out_cps.append(cp_o)

        for k in range(_N_CHUNKS):
            out_cps[k].wait()
            x_rdmas[k].wait_send()
            y_rdmas[k].wait_send()
        for k in range(_N_CHUNKS):
            y_rdmas[k].wait_recv()

    return pl.pallas_call(
        body,
        out_shape=jax.ShapeDtypeStruct((m, n), jnp.float32),
        in_specs=[pl.BlockSpec(memory_space=pl.ANY)],
        out_specs=pl.BlockSpec(memory_space=pltpu.MemorySpace.HBM),
        scratch_shapes=[
            pltpu.VMEM((half, n), jnp.float32),
            pltpu.VMEM((half, n), jnp.float32),
            pltpu.SemaphoreType.DMA((_N_CHUNKS,)),
            pltpu.SemaphoreType.DMA((_N_CHUNKS,)),
            pltpu.SemaphoreType.DMA((_N_CHUNKS,)),
            pltpu.SemaphoreType.DMA((_N_CHUNKS,)),
            pltpu.SemaphoreType.DMA((_N_CHUNKS,)),
            pltpu.SemaphoreType.DMA((_N_CHUNKS,)),
        ],
        compiler_params=pltpu.CompilerParams(collective_id=0),
    )(x)


# device time: 217239 ns/iter; 1.0250x vs baseline; 1.0250x over previous
import jax
import jax.numpy as jnp
from jax import lax
from jax.experimental import pallas as pl
from jax.experimental.pallas import tpu as pltpu

_N_CHUNKS = 32


def kernel(x):
    m, n = x.shape
    half = m // 2
    ch = half // _N_CHUNKS

    def body(
        x_ref,
        out_ref,
        recv_x_ref,
        recv_y_ref,
        a_ref,
        b_ref,
        pa_sems,
        pb_sems,
        oa_sems,
        ob_sems,
        send_sems_x,
        recv_sems_x,
        send_sems_y,
        recv_sems_y,
    ):
        my_x = lax.axis_index("x")
        my_y = lax.axis_index("y")
        x_nbr = (1 - my_x, my_y)
        y_nbr = (my_x, 1 - my_y)

        barrier = pltpu.get_barrier_semaphore()
        for nbr in (x_nbr, y_nbr):
            pl.semaphore_signal(
                barrier, inc=1, device_id=nbr,
                device_id_type=pl.DeviceIdType.MESH,
            )
        pl.semaphore_wait(barrier, 2)

        my_off = my_y * half
        other_off = (1 - my_y) * half

        x_rdmas = []
        for k in range(_N_CHUNKS):
            rdma = pltpu.make_async_remote_copy(
                src_ref=x_ref.at[pl.ds(my_off + k * ch, ch), :],
                dst_ref=recv_x_ref.at[pl.ds(k * ch, ch), :],
                send_sem=send_sems_x.at[k],
                recv_sem=recv_sems_x.at[k],
                device_id=x_nbr,
                device_id_type=pl.DeviceIdType.MESH,
            )
            rdma.start()
            x_rdmas.append(rdma)

        def pf_a(k):
            return pltpu.make_async_copy(
                x_ref.at[pl.ds(my_off + k * ch, ch), :],
                a_ref.at[k % 2],
                pa_sems.at[k % 2],
            )

        def pf_b(j):
            return pltpu.make_async_copy(
                x_ref.at[pl.ds(other_off + j * ch, ch), :],
                b_ref.at[j % 2],
                pb_sems.at[j % 2],
            )

        pf_a(0).start()
        pf_b(0).start()

        y_rdmas = []
        oa_cps = [None] * _N_CHUNKS
        ob_cps = [None] * _N_CHUNKS

        def process_other(j):
            if j + 1 < _N_CHUNKS:
                if j >= 1:
                    ob_cps[j - 1].wait()
                pf_b(j + 1).start()
            pf_b(j).wait()
            y_rdmas[j].wait_recv()
            rows = pl.ds(j * ch, ch)
            b_ref[j % 2] = b_ref[j % 2] + recv_y_ref[rows, :]
            cp = pltpu.make_async_copy(
                b_ref.at[j % 2],
                out_ref.at[pl.ds(other_off + j * ch, ch), :],
                ob_sems.at[j % 2],
            )
            cp.start()
            ob_cps[j] = cp

        for k in range(_N_CHUNKS):
            if k + 1 < _N_CHUNKS:
                if k >= 1:
                    oa_cps[k - 1].wait()
                pf_a(k + 1).start()
            pf_a(k).wait()
            x_rdmas[k].wait_recv()
            rows = pl.ds(k * ch, ch)
            fwd = pltpu.make_async_remote_copy(
                src_ref=recv_x_ref.at[rows, :],
                dst_ref=recv_y_ref.at[rows, :],
                send_sem=send_sems_y.at[k],
                recv_sem=recv_sems_y.at[k],
                device_id=y_nbr,
                device_id_type=pl.DeviceIdType.MESH,
            )
            fwd.start()
            y_rdmas.append(fwd)
            a_ref[k % 2] = a_ref[k % 2] + recv_x_ref[rows, :]
            cp = pltpu.make_async_copy(
                a_ref.at[k % 2],
                out_ref.at[pl.ds(my_off + k * ch, ch), :],
                oa_sems.at[k % 2],
            )
            cp.start()
            oa_cps[k] = cp
            if k >= 1:
                process_other(k - 1)
        process_other(_N_CHUNKS - 1)

        oa_cps[_N_CHUNKS - 2].wait()
        oa_cps[_N_CHUNKS - 1].wait()
        ob_cps[_N_CHUNKS - 2].wait()
        ob_cps[_N_CHUNKS - 1].wait()
        for k in range(_N_CHUNKS):
            x_rdmas[k].wait_send()
            y_rdmas[k].wait_send()

    return pl.pallas_call(
        body,
        out_shape=jax.ShapeDtypeStruct((m, n), jnp.float32),
        in_specs=[pl.BlockSpec(memory_space=pl.ANY)],
        out_specs=pl.BlockSpec(memory_space=pl.ANY),
        scratch_shapes=[
            pltpu.VMEM((half, n), jnp.float32),
            pltpu.VMEM((half, n), jnp.float32),
            pltpu.VMEM((2, ch, n), jnp.float32),
            pltpu.VMEM((2, ch, n), jnp.float32),
            pltpu.SemaphoreType.DMA((2,)),
            pltpu.SemaphoreType.DMA((2,)),
            pltpu.SemaphoreType.DMA((2,)),
            pltpu.SemaphoreType.DMA((2,)),
            pltpu.SemaphoreType.DMA((_N_CHUNKS,)),
            pltpu.SemaphoreType.DMA((_N_CHUNKS,)),
            pltpu.SemaphoreType.DMA((_N_CHUNKS,)),
            pltpu.SemaphoreType.DMA((_N_CHUNKS,)),
        ],
        compiler_params=pltpu.CompilerParams(
            collective_id=0, vmem_limit_bytes=40 * 1024 * 1024
        ),
    )(x)


# device time: 215432 ns/iter; 1.0336x vs baseline; 1.0084x over previous
import jax
import jax.numpy as jnp
from jax import lax
from jax.experimental import pallas as pl
from jax.experimental.pallas import tpu as pltpu

_N_CHUNKS = 64


def kernel(x):
    m, n = x.shape
    half = m // 2
    ch = half // _N_CHUNKS

    def body(
        x_ref,
        out_ref,
        recv_x_ref,
        recv_y_ref,
        a_ref,
        b_ref,
        pa_sems,
        pb_sems,
        oa_sems,
        ob_sems,
        send_sems_x,
        recv_sems_x,
        send_sems_y,
        recv_sems_y,
    ):
        my_x = lax.axis_index("x")
        my_y = lax.axis_index("y")
        x_nbr = (1 - my_x, my_y)
        y_nbr = (my_x, 1 - my_y)

        barrier = pltpu.get_barrier_semaphore()
        for nbr in (x_nbr, y_nbr):
            pl.semaphore_signal(
                barrier, inc=1, device_id=nbr,
                device_id_type=pl.DeviceIdType.MESH,
            )
        pl.semaphore_wait(barrier, 2)

        my_off = my_y * half
        other_off = (1 - my_y) * half

        x_rdmas = []
        for k in range(_N_CHUNKS):
            rdma = pltpu.make_async_remote_copy(
                src_ref=x_ref.at[pl.ds(my_off + k * ch, ch), :],
                dst_ref=recv_x_ref.at[pl.ds(k * ch, ch), :],
                send_sem=send_sems_x.at[k],
                recv_sem=recv_sems_x.at[k],
                device_id=x_nbr,
                device_id_type=pl.DeviceIdType.MESH,
            )
            rdma.start()
            x_rdmas.append(rdma)

        def pf_a(k):
            return pltpu.make_async_copy(
                x_ref.at[pl.ds(my_off + k * ch, ch), :],
                a_ref.at[k % 2],
                pa_sems.at[k % 2],
            )

        def pf_b(j):
            return pltpu.make_async_copy(
                x_ref.at[pl.ds(other_off + j * ch, ch), :],
                b_ref.at[j % 2],
                pb_sems.at[j % 2],
            )

        pf_a(0).start()
        pf_b(0).start()

        y_rdmas = []
        oa_cps = [None] * _N_CHUNKS
        ob_cps = [None] * _N_CHUNKS

        def process_other(j):
            if j + 1 < _N_CHUNKS:
                if j >= 1:
                    ob_cps[j - 1].wait()
                pf_b(j + 1).start()
            pf_b(j).wait()
            y_rdmas[j].wait_recv()
            rows = pl.ds(j * ch, ch)
            b_ref[j % 2] = b_ref[j % 2] + recv_y_ref[rows, :]
            cp = pltpu.make_async_copy(
                b_ref.at[j % 2],
                out_ref.at[pl.ds(other_off + j * ch, ch), :],
                ob_sems.at[j % 2],
            )
            cp.start()
            ob_cps[j] = cp

        for k in range(_N_CHUNKS):
            if k + 1 < _N_CHUNKS:
                if k >= 1:
                    oa_cps[k - 1].wait()
                pf_a(k + 1).start()
            pf_a(k).wait()
            x_rdmas[k].wait_recv()
            rows = pl.ds(k * ch, ch)
            fwd = pltpu.make_async_remote_copy(
                src_ref=recv_x_ref.at[rows, :],
                dst_ref=recv_y_ref.at[rows, :],
                send_sem=send_sems_y.at[k],
                recv_sem=recv_sems_y.at[k],
                device_id=y_nbr,
                device_id_type=pl.DeviceIdType.MESH,
            )
            fwd.start()
            y_rdmas.append(fwd)
            a_ref[k % 2] = a_ref[k % 2] + recv_x_ref[rows, :]
            cp = pltpu.make_async_copy(
                a_ref.at[k % 2],
                out_ref.at[pl.ds(my_off + k * ch, ch), :],
                oa_sems.at[k % 2],
            )
            cp.start()
            oa_cps[k] = cp
            if k >= 1:
                process_other(k - 1)
        process_other(_N_CHUNKS - 1)

        oa_cps[_N_CHUNKS - 2].wait()
        oa_cps[_N_CHUNKS - 1].wait()
        ob_cps[_N_CHUNKS - 2].wait()
        ob_cps[_N_CHUNKS - 1].wait()
        for k in range(_N_CHUNKS):
            x_rdmas[k].wait_send()
            y_rdmas[k].wait_send()

    return pl.pallas_call(
        body,
        out_shape=jax.ShapeDtypeStruct((m, n), jnp.float32),
        in_specs=[pl.BlockSpec(memory_space=pl.ANY)],
        out_specs=pl.BlockSpec(memory_space=pl.ANY),
        scratch_shapes=[
            pltpu.VMEM((half, n), jnp.float32),
            pltpu.VMEM((half, n), jnp.float32),
            pltpu.VMEM((2, ch, n), jnp.float32),
            pltpu.VMEM((2, ch, n), jnp.float32),
            pltpu.SemaphoreType.DMA((2,)),
            pltpu.SemaphoreType.DMA((2,)),
            pltpu.SemaphoreType.DMA((2,)),
            pltpu.SemaphoreType.DMA((2,)),
            pltpu.SemaphoreType.DMA((_N_CHUNKS,)),
            pltpu.SemaphoreType.DMA((_N_CHUNKS,)),
            pltpu.SemaphoreType.DMA((_N_CHUNKS,)),
            pltpu.SemaphoreType.DMA((_N_CHUNKS,)),
        ],
        compiler_params=pltpu.CompilerParams(
            collective_id=0, vmem_limit_bytes=40 * 1024 * 1024
        ),
    )(x)
